# baseline (speedup 1.0000x reference)
import functools

import jax
import jax.numpy as jnp
from jax import lax
from jax.experimental import pallas as pl
from jax.experimental.pallas import tpu as pltpu
from jax.experimental.pallas import tpu_sc as plsc

E = 320000
N = 10000
C = 128
G = 64

NC = 2
NS = 16
NW = NC * NS

CHUNK = 128
NCHUNKS = E // CHUNK
NMAIN = 78
ZROWS = 200
NZ = N // ZROWS
GCH = 80
NGCH = N // GCH
NB = G * N
ZEL = 12800
NZEL = NB // ZEL
HCH = 512
NHCH = E // HCH

_mesh = plsc.VectorSubcoreMesh(core_axis_name="c", subcore_axis_name="s")


@functools.partial(
    pl.kernel,
    out_type=jax.ShapeDtypeStruct((NC, N, C), jnp.float32),
    mesh=_mesh,
    scratch_types=[
        [pltpu.VMEM((CHUNK,), jnp.int32) for _ in range(3)],
        [pltpu.VMEM((CHUNK, C), jnp.float32) for _ in range(3)],
        pltpu.VMEM_SHARED((N, C), jnp.float32),
        [pltpu.SemaphoreType.DMA for _ in range(3)],
        [pltpu.SemaphoreType.DMA for _ in range(3)],
    ],
)
def _agg_kernel(x_hbm, iu_hbm, zrows_hbm, agg_hbm, iu_s, rows_s, agg_sh,
                lsem, ssem):
    cid = lax.axis_index("c")
    sid = lax.axis_index("s")
    wid = sid * NC + cid

    def _issue_loads(slot, c):
        e0 = c * CHUNK
        pltpu.async_copy(iu_hbm.at[pl.ds(e0, CHUNK)], iu_s[slot], lsem[slot])
        pltpu.async_copy(x_hbm.at[pl.ds(e0, CHUNK)], rows_s[slot], lsem[slot])

    _issue_loads(0, wid)
    _issue_loads(1, wid + NW)
    nz_t = NZ // NS + jnp.where(sid < (NZ % NS), 1, 0)

    def zbody(k, _):
        pltpu.sync_copy(zrows_hbm, agg_sh.at[pl.ds((sid + k * NS) * ZROWS, ZROWS)])
        return 0

    lax.fori_loop(0, nz_t, zbody, 0)
    plsc.subcore_barrier()

    def _process(slot):
        pltpu.make_async_copy(iu_hbm.at[pl.ds(0, CHUNK)], iu_s[slot], lsem[slot]).wait()
        pltpu.make_async_copy(x_hbm.at[pl.ds(0, CHUNK)], rows_s[slot], lsem[slot]).wait()
        pltpu.async_copy(rows_s[slot], agg_sh.at[iu_s[slot]], ssem[slot], add=True)

    def _wait_scatter(slot):
        pltpu.make_async_copy(rows_s[slot], agg_sh.at[iu_s[slot]], ssem[slot]).wait()

    def group(g, _):
        for s in range(3):
            k = g * 3 + s
            _process(s)
            s2 = (s + 2) % 3

            @pl.when(k >= 1)
            def _():
                _wait_scatter(s2)

            c2 = wid + (k + 2) * NW

            @pl.when(c2 < NCHUNKS)
            def _():
                _issue_loads(s2, c2)
        return 0

    lax.fori_loop(0, NMAIN // 3, group, 0)
    _wait_scatter(2)

    @pl.when(wid < (NCHUNKS - NMAIN * NW))
    def _():
        _process(0)
        _wait_scatter(0)

    plsc.subcore_barrier()

    def awb(k, _):
        r0 = (sid + k * NS) * ZROWS
        pltpu.async_copy(agg_sh.at[pl.ds(r0, ZROWS)],
                         agg_hbm.at[cid, pl.ds(r0, ZROWS)], lsem[0])
        return 0

    lax.fori_loop(0, nz_t, awb, 0)

    def awb_wait(k, _):
        r0 = (sid + k * NS) * ZROWS
        pltpu.make_async_copy(agg_sh.at[pl.ds(r0, ZROWS)],
                              agg_hbm.at[cid, pl.ds(r0, ZROWS)], lsem[0]).wait()
        return 0

    lax.fori_loop(0, nz_t, awb_wait, 0)


@functools.partial(
    pl.kernel,
    out_type=[
        jax.ShapeDtypeStruct((NB,), jnp.float32),
        jax.ShapeDtypeStruct((NB,), jnp.float32),
        jax.ShapeDtypeStruct((N, C), jnp.float32),
    ],
    mesh=_mesh,
    scratch_types=[
        [pltpu.VMEM((HCH,), jnp.int32) for _ in range(2)],
        [pltpu.VMEM((HCH,), jnp.int32) for _ in range(2)],
        [[pltpu.VMEM((CHUNK,), jnp.int32) for _ in range(HCH // CHUNK)]
         for _ in range(2)],
        pltpu.VMEM((CHUNK,), jnp.float32),
        [pltpu.VMEM((GCH,), jnp.int32) for _ in range(4)],
        [pltpu.VMEM((GCH, C), jnp.float32) for _ in range(4)],
        pltpu.VMEM_SHARED((NB,), jnp.float32),
        [pltpu.SemaphoreType.DMA for _ in range(2)],
        [pltpu.SemaphoreType.DMA for _ in range(2)],
        pltpu.SemaphoreType.DMA,
        pltpu.SemaphoreType.DMA,
        pltpu.SemaphoreType.DMA,
    ],
)
def _hist_kernel(x_hbm, iu_hbm, batch_hbm, ispd_hbm, zel_hbm,
                 cnt0_hbm, cnt1_hbm, h0_hbm,
                 iu_s, bt_s, flat_s, ones_v, gidx_s, grow_s, count_sh,
                 lsem, ssem, gisem, grsem, gosem):
    cid = lax.axis_index("c")
    sid = lax.axis_index("s")
    wid = sid * NC + cid

    nh_t = NHCH // NW + jnp.where(wid < (NHCH % NW), 1, 0)
    ng_t = NGCH // NW + jnp.where(wid < (NGCH % NW), 1, 0)

    def _issue_loads(slot, c):
        e0 = c * HCH
        pltpu.async_copy(iu_hbm.at[pl.ds(e0, HCH)], iu_s[slot], lsem[slot])
        pltpu.async_copy(batch_hbm.at[pl.ds(e0, HCH)], bt_s[slot], lsem[slot])

    def _wait_scats(slot):
        for q in range(HCH // CHUNK):
            pltpu.make_async_copy(
                ones_v, count_sh.at[flat_s[slot][q]], ssem[slot]).wait()

    _issue_loads(0, wid)

    @pl.when(NW + wid < NHCH)
    def _():
        _issue_loads(1, wid + NW)

    for k in range(4):
        @pl.when(k < ng_t)
        def _(k=k):
            r0 = (wid + k * NW) * GCH
            pltpu.async_copy(ispd_hbm.at[pl.ds(r0, GCH)], gidx_s[k], gisem)

    nzel_t = NZEL // NS + jnp.where(sid < (NZEL % NS), 1, 0)

    def zbody(k, _):
        pltpu.sync_copy(zel_hbm, count_sh.at[pl.ds((sid + k * NS) * ZEL, ZEL)])
        return 0

    lax.fori_loop(0, nzel_t, zbody, 0)
    for j in range(CHUNK // 16):
        ones_v[pl.ds(j * 16, 16)] = jnp.ones((16,), jnp.float32)
    plsc.subcore_barrier()

    def visit(k, _):
        for s in range(2):
            ck = 2 * k + s

            @pl.when(ck < nh_t)
            def _():
                @pl.when(ck >= 2)
                def _():
                    _wait_scats(s)

                pltpu.make_async_copy(iu_hbm.at[pl.ds(0, HCH)], iu_s[s], lsem[s]).wait()
                pltpu.make_async_copy(batch_hbm.at[pl.ds(0, HCH)], bt_s[s], lsem[s]).wait()
                for q in range(HCH // CHUNK):
                    for j in range(CHUNK // 16):
                        sl_src = pl.ds(q * CHUNK + j * 16, 16)
                        sl_dst = pl.ds(j * 16, 16)
                        flat_s[s][q][sl_dst] = iu_s[s][sl_src] * G + bt_s[s][sl_src]
                for q in range(HCH // CHUNK):
                    pltpu.async_copy(ones_v, count_sh.at[flat_s[s][q]],
                                     ssem[s], add=True)
                c2 = wid + (ck + 2) * NW

                @pl.when(c2 < NHCH)
                def _():
                    _issue_loads(s, c2)

        return 0

    lax.fori_loop(0, (NHCH // NW + 2) // 2, visit, 0)
    _wait_scats(0)
    _wait_scats(1)

    for k in range(4):
        @pl.when(k < ng_t)
        def _(k=k):
            pltpu.make_async_copy(ispd_hbm.at[pl.ds(0, GCH)], gidx_s[k], gisem).wait()
            pltpu.async_copy(x_hbm.at[gidx_s[k]], grow_s[k], grsem)

    for k in range(4):
        @pl.when(k < ng_t)
        def _(k=k):
            r0 = (wid + k * NW) * GCH
            pltpu.make_async_copy(x_hbm.at[pl.ds(0, GCH)], grow_s[k], grsem).wait()
            pltpu.async_copy(grow_s[k], h0_hbm.at[pl.ds(r0, GCH)], gosem)

    for k in range(4):
        @pl.when(k < ng_t)
        def _(k=k):
            r0 = (wid + k * NW) * GCH
            pltpu.make_async_copy(grow_s[k], h0_hbm.at[pl.ds(r0, GCH)], gosem).wait()

    plsc.subcore_barrier()

    def cwb0(k, _):
        o = (sid + k * NS) * ZEL
        pltpu.async_copy(count_sh.at[pl.ds(o, ZEL)], cnt0_hbm.at[pl.ds(o, ZEL)],
                         lsem[0])
        return 0

    def cwb0_wait(k, _):
        o = (sid + k * NS) * ZEL
        pltpu.make_async_copy(count_sh.at[pl.ds(o, ZEL)],
                              cnt0_hbm.at[pl.ds(o, ZEL)], lsem[0]).wait()
        return 0

    def cwb1(k, _):
        o = (sid + k * NS) * ZEL
        pltpu.async_copy(count_sh.at[pl.ds(o, ZEL)], cnt1_hbm.at[pl.ds(o, ZEL)],
                         lsem[0])
        return 0

    def cwb1_wait(k, _):
        o = (sid + k * NS) * ZEL
        pltpu.make_async_copy(count_sh.at[pl.ds(o, ZEL)],
                              cnt1_hbm.at[pl.ds(o, ZEL)], lsem[0]).wait()
        return 0

    @pl.when(cid == 0)
    def _():
        lax.fori_loop(0, nzel_t, cwb0, 0)
        lax.fori_loop(0, nzel_t, cwb0_wait, 0)

    @pl.when(cid == 1)
    def _():
        lax.fori_loop(0, nzel_t, cwb1, 0)
        lax.fori_loop(0, nzel_t, cwb1_wait, 0)


_MLP_R = 2000


def _fused_body(h0_ref, a0_ref, a1_ref, c0_ref, c1_ref, eps_ref,
                w1_ref, b1_ref, w2_ref, b2_ref, out_ref):
    i = pl.program_id(0)
    scale = 1.0 + eps_ref[0, 0]
    hin = h0_ref[...] * scale + a0_ref[...] + a1_ref[...]
    h1 = jnp.dot(hin, w1_ref[...], preferred_element_type=jnp.float32) + b1_ref[...]
    h1 = jnp.maximum(h1, 0.0)
    h2 = jnp.dot(h1, w2_ref[...], preferred_element_type=jnp.float32) + b2_ref[...]
    h2 = jnp.maximum(h2, 0.0)
    cnt = c0_ref[...] + c1_ref[...]
    contrib = lax.dot_general(cnt, h2, (((0,), (0,)), ((), ())),
                              preferred_element_type=jnp.float32)

    @pl.when(i == 0)
    def _():
        out_ref[...] = contrib

    @pl.when(i > 0)
    def _():
        out_ref[...] += contrib


def _fused_tc(h0, a0, a1, c0, c1, eps, w1t, b1, w2t, b2):
    grid = (N // _MLP_R,)
    return pl.pallas_call(
        _fused_body,
        grid=grid,
        in_specs=[
            pl.BlockSpec((_MLP_R, C), lambda i: (i, 0)),
            pl.BlockSpec((_MLP_R, C), lambda i: (i, 0)),
            pl.BlockSpec((_MLP_R, C), lambda i: (i, 0)),
            pl.BlockSpec((_MLP_R, G), lambda i: (i, 0)),
            pl.BlockSpec((_MLP_R, G), lambda i: (i, 0)),
            pl.BlockSpec((1, 1), lambda i: (0, 0)),
            pl.BlockSpec((C, C), lambda i: (0, 0)),
            pl.BlockSpec((1, C), lambda i: (0, 0)),
            pl.BlockSpec((C, C), lambda i: (0, 0)),
            pl.BlockSpec((1, C), lambda i: (0, 0)),
        ],
        out_specs=pl.BlockSpec((G, C), lambda i: (0, 0)),
        out_shape=jax.ShapeDtypeStruct((G, C), jnp.float32),
    )(h0, a0, a1, c0, c1, eps, w1t, b1, w2t, b2)


def kernel(x, index_u, index_shortest_path_distance, batch, W1, b1, W2, b2, eps):
    zrows = jnp.zeros((ZROWS, C), jnp.float32)
    zel = jnp.zeros((ZEL,), jnp.float32)
    agg2 = _agg_kernel(x, index_u, zrows)
    c0, c1, h0 = _hist_kernel(x, index_u, batch,
                              index_shortest_path_distance, zel)
    return _fused_tc(h0, agg2[0], agg2[1],
                     c0.reshape(N, G), c1.reshape(N, G), eps.reshape(1, 1),
                     W1.T, b1.reshape(1, C), W2.T, b2.reshape(1, C))
